# Initial kernel scaffold; baseline (speedup 1.0000x reference)
import jax
import jax.numpy as jnp
from jax.experimental import pallas as pl


def kernel(user_ids, item_ids, user_emb, item_emb, user_bias, item_bias, global_bias):
    raise NotImplementedError("write your pallas kernel here")


# speedup vs baseline: 1.0182x; 1.0182x over previous
import functools

import jax
import jax.numpy as jnp
from jax import lax
from jax.experimental import pallas as pl
from jax.experimental.pallas import tpu as pltpu
from jax.experimental.pallas import tpu_sc as plsc

NC = 2
NS = 16
L = 16
NW = NC * NS

B = 16384
D = 128
BPW = B // NW
CH = 128
NCHUNK = BPW // CH


def _body(uid_h, iid_h, ue_h, ie_h, ub_h, ib_h, gb_h, out_h,
          uidx, iidx, uev, iev, ubv, ibv, gbv, outv, sem):
    cid = lax.axis_index("c")
    sid = lax.axis_index("s")
    wid = sid * NC + cid
    base = wid * BPW

    pltpu.sync_copy(gb_h, gbv)
    gb = gbv[...]

    for c in range(NCHUNK):
        off = base + c * CH
        pltpu.sync_copy(uid_h.at[pl.ds(off, CH)], uidx)
        pltpu.sync_copy(iid_h.at[pl.ds(off, CH)], iidx)
        cp1 = pltpu.async_copy(ue_h.at[uidx], uev, sem)
        cp2 = pltpu.async_copy(ie_h.at[iidx], iev, sem)
        cp3 = pltpu.async_copy(ub_h.at[uidx], ubv, sem)
        cp4 = pltpu.async_copy(ib_h.at[iidx], ibv, sem)
        cp1.wait()
        cp2.wait()
        cp3.wait()
        cp4.wait()

        lane = lax.iota(jnp.int32, L)

        def group_body(g, carry):
            out16 = jnp.zeros((L,), jnp.float32)
            for r in range(L):
                row = g * L + r
                pacc = uev[row, pl.ds(0, L)] * iev[row, pl.ds(0, L)]
                for j in range(1, D // L):
                    pacc = pacc + (uev[row, pl.ds(j * L, L)] *
                                   iev[row, pl.ds(j * L, L)])
                s = jnp.sum(pacc)
                out16 = jnp.where(lane == r, s, out16)
            ub16 = ubv[pl.ds(g * L, L)]
            ib16 = ibv[pl.ds(g * L, L)]
            outv[pl.ds(c * CH + g * L, L)] = out16 + ub16 + ib16 + gb
            return carry

        lax.fori_loop(0, CH // L, group_body, 0)

    pltpu.sync_copy(outv, out_h.at[pl.ds(base, BPW)])


@functools.partial(jax.jit, static_argnames=())
def kernel(user_ids, item_ids, user_emb, item_emb, user_bias, item_bias,
           global_bias):
    gb16 = jnp.broadcast_to(global_bias.astype(jnp.float32), (L,))
    mesh = plsc.VectorSubcoreMesh(core_axis_name="c", subcore_axis_name="s",
                                  num_cores=NC, num_subcores=NS)
    run = pl.kernel(
        _body,
        out_type=jax.ShapeDtypeStruct((B,), jnp.float32),
        mesh=mesh,
        compiler_params=pltpu.CompilerParams(needs_layout_passes=False),
        scratch_types=[
            pltpu.VMEM((CH,), jnp.int32),
            pltpu.VMEM((CH,), jnp.int32),
            pltpu.VMEM((CH, D), jnp.float32),
            pltpu.VMEM((CH, D), jnp.float32),
            pltpu.VMEM((CH,), jnp.float32),
            pltpu.VMEM((CH,), jnp.float32),
            pltpu.VMEM((L,), jnp.float32),
            pltpu.VMEM((BPW,), jnp.float32),
            pltpu.SemaphoreType.DMA,
        ],
    )
    return run(user_ids.astype(jnp.int32), item_ids.astype(jnp.int32),
               user_emb, item_emb, user_bias, item_bias, gb16)


# speedup vs baseline: 1.2015x; 1.1800x over previous
import functools

import jax
import jax.numpy as jnp
from jax import lax
from jax.experimental import pallas as pl
from jax.experimental.pallas import tpu as pltpu
from jax.experimental.pallas import tpu_sc as plsc

NC = 2
NS = 16
L = 16
NW = NC * NS

B = 16384
D = 128
BPW = B // NW
CH = 64
NCHUNK = BPW // CH
NBUF = 2


def _body(uid_h, iid_h, ue_h, ie_h, ub_h, ib_h, gb_h, out_h,
          uidx_all, iidx_all, uev, iev, ubv, ibv, gbv, outv,
          sem0, sem1, semb):
    cid = lax.axis_index("c")
    sid = lax.axis_index("s")
    wid = sid * NC + cid
    base = wid * BPW
    sems = [sem0, sem1]
    lane = lax.iota(jnp.int32, L)

    pltpu.sync_copy(gb_h, gbv)
    gb = gbv[...]

    pltpu.sync_copy(uid_h.at[pl.ds(base, BPW)], uidx_all)
    pltpu.sync_copy(iid_h.at[pl.ds(base, BPW)], iidx_all)

    bias_cps = []
    for q in range(BPW // 128):
        sl = pl.ds(q * 128, 128)
        bias_cps.append(pltpu.async_copy(ub_h.at[uidx_all.at[sl]],
                                         ubv.at[sl], semb))
        bias_cps.append(pltpu.async_copy(ib_h.at[iidx_all.at[sl]],
                                         ibv.at[sl], semb))

    def issue(c, b):
        isl = pl.ds(c * CH, CH)
        pltpu.async_copy(ue_h.at[uidx_all.at[isl]], uev.at[b], sems[b])
        pltpu.async_copy(ie_h.at[iidx_all.at[isl]], iev.at[b], sems[b])

    def drain(c, b):
        isl = pl.ds(c * CH, CH)
        pltpu.make_async_copy(ue_h.at[uidx_all.at[isl]], uev.at[b],
                              sems[b]).wait()
        pltpu.make_async_copy(ie_h.at[iidx_all.at[isl]], iev.at[b],
                              sems[b]).wait()

    issue(0, 0)
    issue(1, 1)
    for cp in bias_cps:
        cp.wait()

    def iter_body(k, carry):
        for b in range(NBUF):
            c = k * NBUF + b
            drain(c, b)
            for g in range(CH // L):
                out16 = jnp.zeros((L,), jnp.float32)
                for r in range(L):
                    row = g * L + r
                    pacc = (uev[b, row, pl.ds(0, L)] *
                            iev[b, row, pl.ds(0, L)])
                    for j in range(1, D // L):
                        pacc = pacc + (uev[b, row, pl.ds(j * L, L)] *
                                       iev[b, row, pl.ds(j * L, L)])
                    s = jnp.sum(pacc)
                    out16 = jnp.where(lane == r, s, out16)
                off = c * CH + g * L
                ub16 = ubv[pl.ds(off, L)]
                ib16 = ibv[pl.ds(off, L)]
                outv[pl.ds(off, L)] = out16 + ub16 + ib16 + gb

            @pl.when(c + NBUF < NCHUNK)
            def _():
                issue(c + NBUF, b)
        return carry

    lax.fori_loop(0, NCHUNK // NBUF, iter_body, 0)

    pltpu.sync_copy(outv, out_h.at[pl.ds(base, BPW)])


@functools.partial(jax.jit, static_argnames=())
def kernel(user_ids, item_ids, user_emb, item_emb, user_bias, item_bias,
           global_bias):
    gb16 = jnp.broadcast_to(global_bias.astype(jnp.float32), (L,))
    mesh = plsc.VectorSubcoreMesh(core_axis_name="c", subcore_axis_name="s",
                                  num_cores=NC, num_subcores=NS)
    run = pl.kernel(
        _body,
        out_type=jax.ShapeDtypeStruct((B,), jnp.float32),
        mesh=mesh,
        compiler_params=pltpu.CompilerParams(needs_layout_passes=False),
        scratch_types=[
            pltpu.VMEM((BPW,), jnp.int32),
            pltpu.VMEM((BPW,), jnp.int32),
            pltpu.VMEM((NBUF, CH, D), jnp.float32),
            pltpu.VMEM((NBUF, CH, D), jnp.float32),
            pltpu.VMEM((BPW,), jnp.float32),
            pltpu.VMEM((BPW,), jnp.float32),
            pltpu.VMEM((L,), jnp.float32),
            pltpu.VMEM((BPW,), jnp.float32),
            pltpu.SemaphoreType.DMA,
            pltpu.SemaphoreType.DMA,
            pltpu.SemaphoreType.DMA,
        ],
    )
    return run(user_ids.astype(jnp.int32), item_ids.astype(jnp.int32),
               user_emb, item_emb, user_bias, item_bias, gb16)


# speedup vs baseline: 1.2601x; 1.0488x over previous
import functools

import jax
import jax.numpy as jnp
from jax import lax
from jax.experimental import pallas as pl
from jax.experimental.pallas import tpu as pltpu
from jax.experimental.pallas import tpu_sc as plsc

NC = 2
NS = 16
L = 16
NW = NC * NS

B = 16384
D = 128
BPW = B // NW
CH = 64
NCHUNK = BPW // CH
NBUF = 2


def _body(uid_h, iid_h, ue_h, ie_h, ub_h, ib_h, gb_h, out_h,
          uidx_all, iidx_all, uev, iev, ubv, ibv, gbv, outv,
          sem0, sem1, semb):
    cid = lax.axis_index("c")
    sid = lax.axis_index("s")
    wid = sid * NC + cid
    base = wid * BPW
    sems = [sem0, sem1]
    lane = lax.iota(jnp.int32, L)

    pltpu.sync_copy(gb_h, gbv)

    pltpu.sync_copy(uid_h.at[pl.ds(base, BPW)], uidx_all)
    pltpu.sync_copy(iid_h.at[pl.ds(base, BPW)], iidx_all)

    def issue(c, b):
        isl = pl.ds(c * CH, CH)
        pltpu.async_copy(ue_h.at[uidx_all.at[isl]], uev.at[b], sems[b])
        pltpu.async_copy(ie_h.at[iidx_all.at[isl]], iev.at[b], sems[b])

    def drain(c, b):
        isl = pl.ds(c * CH, CH)
        pltpu.make_async_copy(ue_h.at[uidx_all.at[isl]], uev.at[b],
                              sems[b]).wait()
        pltpu.make_async_copy(ie_h.at[iidx_all.at[isl]], iev.at[b],
                              sems[b]).wait()

    issue(0, 0)
    issue(1, 1)

    bias_cps = []
    for q in range(BPW // 128):
        sl = pl.ds(q * 128, 128)
        bias_cps.append(pltpu.async_copy(ub_h.at[uidx_all.at[sl]],
                                         ubv.at[sl], semb))
        bias_cps.append(pltpu.async_copy(ib_h.at[iidx_all.at[sl]],
                                         ibv.at[sl], semb))

    def iter_body(k, carry):
        for b in range(NBUF):
            c = k * NBUF + b
            drain(c, b)

            def group_body(g, carry2):
                out16 = jnp.zeros((L,), jnp.float32)
                for r in range(L):
                    row = g * L + r
                    pacc = (uev[b, row, pl.ds(0, L)] *
                            iev[b, row, pl.ds(0, L)])
                    for j in range(1, D // L):
                        pacc = pacc + (uev[b, row, pl.ds(j * L, L)] *
                                       iev[b, row, pl.ds(j * L, L)])
                    s = jnp.sum(pacc)
                    out16 = jnp.where(lane == r, s, out16)
                outv[pl.ds(c * CH + g * L, L)] = out16
                return carry2

            lax.fori_loop(0, CH // L, group_body, 0)

            @pl.when(c + NBUF < NCHUNK)
            def _():
                issue(c + NBUF, b)
        return carry

    lax.fori_loop(0, NCHUNK // NBUF, iter_body, 0)

    for cp in bias_cps:
        cp.wait()
    gb = gbv[...]

    def bias_body(g, carry):
        sl = pl.ds(g * L, L)
        outv[sl] = outv[sl] + ubv[sl] + ibv[sl] + gb
        return carry

    lax.fori_loop(0, BPW // L, bias_body, 0)

    pltpu.sync_copy(outv, out_h.at[pl.ds(base, BPW)])


@functools.partial(jax.jit, static_argnames=())
def kernel(user_ids, item_ids, user_emb, item_emb, user_bias, item_bias,
           global_bias):
    gb16 = jnp.broadcast_to(global_bias.astype(jnp.float32), (L,))
    mesh = plsc.VectorSubcoreMesh(core_axis_name="c", subcore_axis_name="s",
                                  num_cores=NC, num_subcores=NS)
    run = pl.kernel(
        _body,
        out_type=jax.ShapeDtypeStruct((B,), jnp.float32),
        mesh=mesh,
        compiler_params=pltpu.CompilerParams(needs_layout_passes=False),
        scratch_types=[
            pltpu.VMEM((BPW,), jnp.int32),
            pltpu.VMEM((BPW,), jnp.int32),
            pltpu.VMEM((NBUF, CH, D), jnp.float32),
            pltpu.VMEM((NBUF, CH, D), jnp.float32),
            pltpu.VMEM((BPW,), jnp.float32),
            pltpu.VMEM((BPW,), jnp.float32),
            pltpu.VMEM((L,), jnp.float32),
            pltpu.VMEM((BPW,), jnp.float32),
            pltpu.SemaphoreType.DMA,
            pltpu.SemaphoreType.DMA,
            pltpu.SemaphoreType.DMA,
        ],
    )
    return run(user_ids.astype(jnp.int32), item_ids.astype(jnp.int32),
               user_emb, item_emb, user_bias, item_bias, gb16)


# speedup vs baseline: 1.8118x; 1.4378x over previous
import functools

import jax
import jax.numpy as jnp
from jax import lax
from jax.experimental import pallas as pl
from jax.experimental.pallas import tpu as pltpu
from jax.experimental.pallas import tpu_sc as plsc

NC = 2
NS = 16
L = 16
NW = NC * NS

B = 16384
D = 128
BPW = B // NW
CH = 128
NCHUNK = BPW // CH
NBUF = 2
RUNROLL = 4


def _body(uid_h, iid_h, ue_h, ie_h, ub_h, ib_h, gb_h, out_h,
          uidx_all, iidx_all, uev, iev, ubv, ibv, gbs, outv,
          sem0, sem1, semb):
    cid = lax.axis_index("c")
    sid = lax.axis_index("s")
    wid = sid * NC + cid
    base = wid * BPW
    sems = [sem0, sem1]
    lane = lax.iota(jnp.int32, L)

    pltpu.sync_copy(gb_h, gbs)

    pltpu.sync_copy(uid_h.at[pl.ds(base, BPW)], uidx_all)
    pltpu.sync_copy(iid_h.at[pl.ds(base, BPW)], iidx_all)

    def issue(c, b):
        isl = pl.ds(c * CH, CH)
        pltpu.async_copy(ue_h.at[uidx_all.at[isl]], uev.at[b], sems[b])
        pltpu.async_copy(ie_h.at[iidx_all.at[isl]], iev.at[b], sems[b])

    def drain(c, b):
        isl = pl.ds(c * CH, CH)
        pltpu.make_async_copy(ue_h.at[uidx_all.at[isl]], uev.at[b],
                              sems[b]).wait()
        pltpu.make_async_copy(ie_h.at[iidx_all.at[isl]], iev.at[b],
                              sems[b]).wait()

    issue(0, 0)
    issue(1, 1)

    bias_cps = []
    for q in range(BPW // 128):
        sl = pl.ds(q * 128, 128)
        bias_cps.append(pltpu.async_copy(ub_h.at[uidx_all.at[sl]],
                                         ubv.at[sl], semb))
        bias_cps.append(pltpu.async_copy(ib_h.at[iidx_all.at[sl]],
                                         ibv.at[sl], semb))

    def iter_body(k, carry):
        for b in range(NBUF):
            c = k * NBUF + b
            drain(c, b)

            def group_body(g, carry2):
                def sub_body(rr, out16):
                    for q in range(RUNROLL):
                        r = rr * RUNROLL + q
                        row = g * L + r
                        pacc = (uev[b, row, pl.ds(0, L)] *
                                iev[b, row, pl.ds(0, L)])
                        for j in range(1, D // L):
                            pacc = pacc + (uev[b, row, pl.ds(j * L, L)] *
                                           iev[b, row, pl.ds(j * L, L)])
                        s = jnp.sum(pacc)
                        out16 = jnp.where(lane == r, s, out16)
                    return out16

                out16 = lax.fori_loop(0, L // RUNROLL, sub_body,
                                      jnp.zeros((L,), jnp.float32))
                outv[pl.ds(c * CH + g * L, L)] = out16
                return carry2

            lax.fori_loop(0, CH // L, group_body, 0)

            @pl.when(c + NBUF < NCHUNK)
            def _():
                issue(c + NBUF, b)
        return carry

    lax.fori_loop(0, NCHUNK // NBUF, iter_body, 0)

    for cp in bias_cps:
        cp.wait()
    gb = gbs[...]

    def bias_body(g, carry):
        sl = pl.ds(g * L, L)
        outv[sl] = outv[sl] + ubv[sl] + ibv[sl] + gb
        return carry

    lax.fori_loop(0, BPW // L, bias_body, 0)

    pltpu.sync_copy(outv, out_h.at[pl.ds(base, BPW)])


@functools.partial(jax.jit, static_argnames=())
def kernel(user_ids, item_ids, user_emb, item_emb, user_bias, item_bias,
           global_bias):
    gb1 = jnp.broadcast_to(global_bias.astype(jnp.float32), (L,))
    mesh = plsc.VectorSubcoreMesh(core_axis_name="c", subcore_axis_name="s",
                                  num_cores=NC, num_subcores=NS)
    run = pl.kernel(
        _body,
        out_type=jax.ShapeDtypeStruct((B,), jnp.float32),
        mesh=mesh,
        compiler_params=pltpu.CompilerParams(needs_layout_passes=False),
        scratch_types=[
            pltpu.VMEM((BPW,), jnp.int32),
            pltpu.VMEM((BPW,), jnp.int32),
            pltpu.VMEM((NBUF, CH, D), jnp.float32),
            pltpu.VMEM((NBUF, CH, D), jnp.float32),
            pltpu.VMEM((BPW,), jnp.float32),
            pltpu.VMEM((BPW,), jnp.float32),
            pltpu.VMEM((L,), jnp.float32),
            pltpu.VMEM((BPW,), jnp.float32),
            pltpu.SemaphoreType.DMA,
            pltpu.SemaphoreType.DMA,
            pltpu.SemaphoreType.DMA,
        ],
    )
    return run(user_ids.astype(jnp.int32), item_ids.astype(jnp.int32),
               user_emb, item_emb, user_bias, item_bias, gb1)


# speedup vs baseline: 1.8254x; 1.0075x over previous
import functools

import jax
import jax.numpy as jnp
from jax import lax
from jax.experimental import pallas as pl
from jax.experimental.pallas import tpu as pltpu
from jax.experimental.pallas import tpu_sc as plsc

NC = 2
NS = 16
L = 16
NW = NC * NS

B = 16384
D = 128
BPW = B // NW
CH = 128
CHUNKS = (64, 128, 128, 128, 64)
OFFS = (0, 64, 192, 320, 448)
NBUF = 2
RUNROLL = 4


def _body(uid_h, iid_h, ue_h, ie_h, ub_h, ib_h, gb_h, out_h,
          uidx_all, iidx_all, uev, iev, ubv, ibv, gbs, outv,
          sem0, sem1, semb):
    cid = lax.axis_index("c")
    sid = lax.axis_index("s")
    wid = sid * NC + cid
    base = wid * BPW
    sems = [sem0, sem1]
    lane = lax.iota(jnp.int32, L)

    pltpu.sync_copy(gb_h, gbs)

    pltpu.sync_copy(uid_h.at[pl.ds(base, BPW)], uidx_all)
    pltpu.sync_copy(iid_h.at[pl.ds(base, BPW)], iidx_all)

    def issue(c, b):
        isl = pl.ds(OFFS[c], CHUNKS[c])
        dsl = pl.ds(0, CHUNKS[c])
        pltpu.async_copy(ue_h.at[uidx_all.at[isl]], uev.at[b, dsl], sems[b])
        pltpu.async_copy(ie_h.at[iidx_all.at[isl]], iev.at[b, dsl], sems[b])

    def drain(c, b):
        isl = pl.ds(OFFS[c], CHUNKS[c])
        dsl = pl.ds(0, CHUNKS[c])
        pltpu.make_async_copy(ue_h.at[uidx_all.at[isl]], uev.at[b, dsl],
                              sems[b]).wait()
        pltpu.make_async_copy(ie_h.at[iidx_all.at[isl]], iev.at[b, dsl],
                              sems[b]).wait()

    issue(0, 0)
    issue(1, 1)

    bias_cps = []
    for q in range(BPW // 128):
        sl = pl.ds(q * 128, 128)
        bias_cps.append(pltpu.async_copy(ub_h.at[uidx_all.at[sl]],
                                         ubv.at[sl], semb))
        bias_cps.append(pltpu.async_copy(ib_h.at[iidx_all.at[sl]],
                                         ibv.at[sl], semb))

    for c in range(len(CHUNKS)):
        b = c % NBUF
        drain(c, b)

        def group_body(g, carry2, b=b, c=c):
            def sub_body(rr, out16):
                for q in range(RUNROLL):
                    r = rr * RUNROLL + q
                    row = g * L + r
                    pacc = (uev[b, row, pl.ds(0, L)] *
                            iev[b, row, pl.ds(0, L)])
                    for j in range(1, D // L):
                        pacc = pacc + (uev[b, row, pl.ds(j * L, L)] *
                                       iev[b, row, pl.ds(j * L, L)])
                    s = jnp.sum(pacc)
                    out16 = jnp.where(lane == r, s, out16)
                return out16

            out16 = lax.fori_loop(0, L // RUNROLL, sub_body,
                                  jnp.zeros((L,), jnp.float32))
            outv[pl.ds(OFFS[c] + g * L, L)] = out16
            return carry2

        lax.fori_loop(0, CHUNKS[c] // L, group_body, 0)

        if c + NBUF < len(CHUNKS):
            issue(c + NBUF, b)

    for cp in bias_cps:
        cp.wait()
    gb = gbs[...]

    def bias_body(g, carry):
        sl = pl.ds(g * L, L)
        outv[sl] = outv[sl] + ubv[sl] + ibv[sl] + gb
        return carry

    lax.fori_loop(0, BPW // L, bias_body, 0)

    pltpu.sync_copy(outv, out_h.at[pl.ds(base, BPW)])


@functools.partial(jax.jit, static_argnames=())
def kernel(user_ids, item_ids, user_emb, item_emb, user_bias, item_bias,
           global_bias):
    gb1 = jnp.broadcast_to(global_bias.astype(jnp.float32), (L,))
    mesh = plsc.VectorSubcoreMesh(core_axis_name="c", subcore_axis_name="s",
                                  num_cores=NC, num_subcores=NS)
    run = pl.kernel(
        _body,
        out_type=jax.ShapeDtypeStruct((B,), jnp.float32),
        mesh=mesh,
        compiler_params=pltpu.CompilerParams(needs_layout_passes=False),
        scratch_types=[
            pltpu.VMEM((BPW,), jnp.int32),
            pltpu.VMEM((BPW,), jnp.int32),
            pltpu.VMEM((NBUF, CH, D), jnp.float32),
            pltpu.VMEM((NBUF, CH, D), jnp.float32),
            pltpu.VMEM((BPW,), jnp.float32),
            pltpu.VMEM((BPW,), jnp.float32),
            pltpu.VMEM((L,), jnp.float32),
            pltpu.VMEM((BPW,), jnp.float32),
            pltpu.SemaphoreType.DMA,
            pltpu.SemaphoreType.DMA,
            pltpu.SemaphoreType.DMA,
        ],
    )
    return run(user_ids.astype(jnp.int32), item_ids.astype(jnp.int32),
               user_emb, item_emb, user_bias, item_bias, gb1)
